# Initial kernel scaffold; baseline (speedup 1.0000x reference)
import functools

import jax
import jax.numpy as jnp
from jax import lax
from jax.experimental import pallas as pl
from jax.experimental.pallas import tpu as pltpu
from jax.experimental.pallas import tpu_sc as plsc

VOCAB = 1000000
DIM = 32
B = 4096
L = 200

_INFO = plsc.get_sparse_core_info()
_NC = _INFO.num_cores
_NS = _INFO.num_subcores
_NW = _NC * _NS

_N = B * L
_CHUNK = 128
_PER_W = _N // _NW
_NCHUNK = _PER_W // _CHUNK

_mesh = plsc.VectorSubcoreMesh(core_axis_name="c", subcore_axis_name="s")


@functools.partial(
    pl.kernel,
    mesh=_mesh,
    out_type=jax.ShapeDtypeStruct((_N, DIM), jnp.float32),
    scratch_types=[
        pltpu.VMEM((_NCHUNK, _CHUNK), jnp.int32),
        pltpu.VMEM((2, _CHUNK, DIM), jnp.float32),
        pltpu.SemaphoreType.DMA,
        pltpu.SemaphoreType.DMA,
    ],
)
def _gather(weight_hbm, idx_hbm, out_hbm, idx_v, rows_v, gsem, ssem):
    wid = lax.axis_index("s") * _NC + lax.axis_index("c")
    base_chunk = wid * _NCHUNK
    base_row = wid * _PER_W

    pltpu.sync_copy(idx_hbm.at[pl.ds(base_chunk, _NCHUNK)], idx_v)

    def gather_start(j, buf):
        return pltpu.async_copy(weight_hbm.at[idx_v.at[j]], rows_v.at[buf], gsem)

    gather_start(0, 0)

    def body(j, _):
        buf = lax.rem(j, 2)
        pltpu.make_async_copy(weight_hbm.at[idx_v.at[j]], rows_v.at[buf], gsem).wait()
        @pl.when(j + 1 < _NCHUNK)
        def _():
            gather_start(j + 1, 1 - buf)
        store = pltpu.async_copy(
            rows_v.at[buf], out_hbm.at[pl.ds(base_row + j * _CHUNK, _CHUNK)], ssem)
        store.wait()
        return 0

    lax.fori_loop(0, _NCHUNK, body, 0)


def kernel(tensor, weight):
    idx_t = tensor.T.reshape(_N // _CHUNK, _CHUNK)
    out = _gather(weight, idx_t)
    return out.reshape(L, B, DIM)


# speedup vs baseline: 1.4132x; 1.4132x over previous
import functools

import jax
import jax.numpy as jnp
from jax import lax
from jax.experimental import pallas as pl
from jax.experimental.pallas import tpu as pltpu
from jax.experimental.pallas import tpu_sc as plsc

VOCAB = 1000000
DIM = 32
B = 4096
L = 200

_INFO = plsc.get_sparse_core_info()
_NC = _INFO.num_cores
_NS = _INFO.num_subcores
_NW = _NC * _NS

_N = B * L
_CHUNK = 128
_PER_W = _N // _NW
_NCHUNK = _PER_W // _CHUNK

_mesh = plsc.VectorSubcoreMesh(core_axis_name="c", subcore_axis_name="s")


@functools.partial(
    pl.kernel,
    mesh=_mesh,
    out_type=jax.ShapeDtypeStruct((_N, DIM), jnp.float32),
    scratch_types=[
        pltpu.VMEM((_NCHUNK, _CHUNK), jnp.int32),
        pltpu.VMEM((2, _CHUNK, DIM), jnp.float32),
        pltpu.SemaphoreType.DMA,
        pltpu.SemaphoreType.DMA,
    ],
    compiler_params=pltpu.CompilerParams(use_tc_tiling_on_sc=False),
)
def _gather(weight_hbm, idx_hbm, out_hbm, idx_v, rows_v, gsem, ssem):
    wid = lax.axis_index("s") * _NC + lax.axis_index("c")
    base_chunk = wid * _NCHUNK
    base_row = wid * _PER_W

    pltpu.sync_copy(idx_hbm.at[pl.ds(base_chunk, _NCHUNK)], idx_v)

    def gather_start(j, buf):
        return pltpu.async_copy(weight_hbm.at[idx_v.at[j]], rows_v.at[buf], gsem)

    gather_start(0, 0)

    def body(j, _):
        buf = lax.rem(j, 2)
        pltpu.make_async_copy(weight_hbm.at[idx_v.at[j]], rows_v.at[buf], gsem).wait()
        @pl.when(j + 1 < _NCHUNK)
        def _():
            gather_start(j + 1, 1 - buf)
        store = pltpu.async_copy(
            rows_v.at[buf], out_hbm.at[pl.ds(base_row + j * _CHUNK, _CHUNK)], ssem)
        store.wait()
        return 0

    lax.fori_loop(0, _NCHUNK, body, 0)


def kernel(tensor, weight):
    idx_t = tensor.T.reshape(_N // _CHUNK, _CHUNK)
    out = _gather(weight, idx_t)
    return out.reshape(L, B, DIM)
